# baseline (speedup 1.0000x reference)
import functools

import jax
import jax.numpy as jnp
from jax import lax
from jax.experimental import pallas as pl
from jax.experimental.pallas import tpu as pltpu
from jax.experimental.pallas import tpu_sc as plsc

_NC = 2
_NS = 16
_L = 16
_NW = _NC * _NS

_CHUNK = 8192


def _make_sc_call(total: int, hi: float):
    per_w = total // _NW
    n_chunks = per_w // _CHUNK
    mesh = plsc.VectorSubcoreMesh(
        core_axis_name="c", subcore_axis_name="s",
        num_cores=_NC, num_subcores=_NS)

    @functools.partial(
        pl.kernel,
        out_type=jax.ShapeDtypeStruct((total,), jnp.float32),
        mesh=mesh,
        compiler_params=pltpu.CompilerParams(needs_layout_passes=False),
        scratch_types=[
            pltpu.VMEM((32,), jnp.float32),
            pltpu.VMEM((32,), jnp.float32),
            pltpu.VMEM((32,), jnp.float32),
            pltpu.VMEM((32,), jnp.float32),
            pltpu.VMEM((_CHUNK,), jnp.float32),
            pltpu.VMEM((_CHUNK,), jnp.float32),
            pltpu.VMEM((_CHUNK,), jnp.float32),
        ],
    )
    def sc_kernel(x_hbm, e_hbm, m_hbm, dm_hbm, s_hbm, ds_hbm, o_hbm,
                  m_v, dm_v, s_v, ds_v, x_v, e_v, o_v):
        wid = lax.axis_index("s") * _NC + lax.axis_index("c")
        base = wid * per_w
        pltpu.sync_copy(m_hbm, m_v)
        pltpu.sync_copy(dm_hbm, dm_v)
        pltpu.sync_copy(s_hbm, s_v)
        pltpu.sync_copy(ds_hbm, ds_v)

        def chunk_body(ci, _):
            off = base + ci * _CHUNK
            pltpu.sync_copy(x_hbm.at[pl.ds(off, _CHUNK)], x_v)
            pltpu.sync_copy(e_hbm.at[pl.ds(off, _CHUNK)], e_v)

            def vec_body(i, _):
                xv = x_v[pl.ds(i * _L, _L)]
                t = xv * hi
                t = jnp.minimum(jnp.maximum(t, 0.0), hi)
                idx = t.astype(jnp.int32)
                a = t - idx.astype(jnp.float32)
                m = plsc.load_gather(m_v, [idx])
                dm = plsc.load_gather(dm_v, [idx])
                s = plsc.load_gather(s_v, [idx])
                ds = plsc.load_gather(ds_v, [idx])
                ev = e_v[pl.ds(i * _L, _L)]
                o_v[pl.ds(i * _L, _L)] = (m + a * dm) + (s + a * ds) * ev
                return 0

            lax.fori_loop(0, _CHUNK // _L, vec_body, 0)
            pltpu.sync_copy(o_v, o_hbm.at[pl.ds(off, _CHUNK)])
            return 0

        lax.fori_loop(0, n_chunks, chunk_body, 0)

    return sc_kernel


def kernel(x, means, stds, epsilon):
    shape = x.shape
    total = x.size
    num_levels = means.shape[0]
    scale = 1.0 / 4095.0
    m = means.astype(jnp.float32) * scale
    s = stds.astype(jnp.float32) * scale
    zero = jnp.zeros((1,), jnp.float32)
    dm = jnp.concatenate([m[1:] - m[:-1], zero])
    ds = jnp.concatenate([s[1:] - s[:-1], zero])
    pad = jnp.zeros((32 - num_levels,), jnp.float32)
    m = jnp.concatenate([m, pad])
    dm = jnp.concatenate([dm, pad])
    s = jnp.concatenate([s, pad])
    ds = jnp.concatenate([ds, pad])
    xf = x.reshape(total)
    ef = epsilon.reshape(total)
    out = _make_sc_call(total, float(num_levels - 1))(xf, ef, m, dm, s, ds)
    return out.reshape(shape)


# speedup vs baseline: 2.1761x; 2.1761x over previous
import functools

import jax
import jax.numpy as jnp
from jax import lax
from jax.experimental import pallas as pl
from jax.experimental.pallas import tpu as pltpu
from jax.experimental.pallas import tpu_sc as plsc

_NC = 2
_NS = 16
_L = 16
_NW = _NC * _NS

_CHUNK = 16384
_UNROLL = 8


def _make_sc_call(total: int, hi: float):
    per_w = total // _NW
    n_chunks = per_w // _CHUNK
    mesh = plsc.VectorSubcoreMesh(
        core_axis_name="c", subcore_axis_name="s",
        num_cores=_NC, num_subcores=_NS)

    @functools.partial(
        pl.kernel,
        out_type=jax.ShapeDtypeStruct((total,), jnp.float32),
        mesh=mesh,
        compiler_params=pltpu.CompilerParams(needs_layout_passes=False),
        scratch_types=[
            pltpu.VMEM((32,), jnp.float32),
            pltpu.VMEM((32,), jnp.float32),
            pltpu.VMEM((32,), jnp.float32),
            pltpu.VMEM((32,), jnp.float32),
            pltpu.VMEM((_CHUNK,), jnp.float32),
            pltpu.VMEM((_CHUNK,), jnp.float32),
            pltpu.VMEM((_CHUNK,), jnp.float32),
            pltpu.VMEM((_CHUNK,), jnp.float32),
            pltpu.VMEM((_CHUNK,), jnp.float32),
            pltpu.VMEM((_CHUNK,), jnp.float32),
            pltpu.SemaphoreType.DMA,
            pltpu.SemaphoreType.DMA,
            pltpu.SemaphoreType.DMA,
            pltpu.SemaphoreType.DMA,
            pltpu.SemaphoreType.DMA,
            pltpu.SemaphoreType.DMA,
        ],
    )
    def sc_kernel(x_hbm, e_hbm, m_hbm, dm_hbm, s_hbm, ds_hbm, o_hbm,
                  m_v, dm_v, s_v, ds_v,
                  x0, x1, e0, e1, o0, o1,
                  sx0, sx1, se0, se1, so0, so1):
        wid = lax.axis_index("s") * _NC + lax.axis_index("c")
        base = wid * per_w
        pltpu.sync_copy(m_hbm, m_v)
        pltpu.sync_copy(dm_hbm, dm_v)
        pltpu.sync_copy(s_hbm, s_v)
        pltpu.sync_copy(ds_hbm, ds_v)

        xb, eb, ob = (x0, x1), (e0, e1), (o0, o1)
        sx, se, so = (sx0, sx1), (se0, se1), (so0, so1)

        def cin(ci, b):
            off = base + ci * _CHUNK
            return (
                pltpu.make_async_copy(
                    x_hbm.at[pl.ds(off, _CHUNK)], xb[b], sx[b]),
                pltpu.make_async_copy(
                    e_hbm.at[pl.ds(off, _CHUNK)], eb[b], se[b]),
            )

        def cout(ci, b):
            off = base + ci * _CHUNK
            return pltpu.make_async_copy(
                ob[b], o_hbm.at[pl.ds(off, _CHUNK)], so[b])

        def compute(xr, er, outr):
            @plsc.parallel_loop(0, _CHUNK, step=_L, unroll=_UNROLL)
            def _body(i):
                xv = xr[pl.ds(i, _L)]
                t = xv * hi
                t = jnp.minimum(jnp.maximum(t, 0.0), hi)
                idx = t.astype(jnp.int32)
                a = t - idx.astype(jnp.float32)
                m = plsc.load_gather(m_v, [idx])
                dm = plsc.load_gather(dm_v, [idx])
                s = plsc.load_gather(s_v, [idx])
                ds = plsc.load_gather(ds_v, [idx])
                ev = er[pl.ds(i, _L)]
                outr[pl.ds(i, _L)] = (m + a * dm) + (s + a * ds) * ev

        for d in cin(0, 0):
            d.start()

        def outer(g, _):
            for b in range(2):
                ci = 2 * g + b

                @pl.when(ci + 1 < n_chunks)
                def _():
                    for d in cin(ci + 1, 1 - b):
                        d.start()

                for d in cin(ci, b):
                    d.wait()

                @pl.when(ci >= 2)
                def _():
                    cout(ci - 2, b).wait()

                compute(xb[b], eb[b], ob[b])
                cout(ci, b).start()
            return 0

        lax.fori_loop(0, n_chunks // 2, outer, 0)
        cout(n_chunks - 2, 0).wait()
        cout(n_chunks - 1, 1).wait()

    return sc_kernel


def kernel(x, means, stds, epsilon):
    shape = x.shape
    total = x.size
    num_levels = means.shape[0]
    scale = 1.0 / 4095.0
    m = means.astype(jnp.float32) * scale
    s = stds.astype(jnp.float32) * scale
    zero = jnp.zeros((1,), jnp.float32)
    dm = jnp.concatenate([m[1:] - m[:-1], zero])
    ds = jnp.concatenate([s[1:] - s[:-1], zero])
    pad = jnp.zeros((32 - num_levels,), jnp.float32)
    m = jnp.concatenate([m, pad])
    dm = jnp.concatenate([dm, pad])
    s = jnp.concatenate([s, pad])
    ds = jnp.concatenate([ds, pad])
    xf = x.reshape(total)
    ef = epsilon.reshape(total)
    out = _make_sc_call(total, float(num_levels - 1))(xf, ef, m, dm, s, ds)
    return out.reshape(shape)


# speedup vs baseline: 2.4009x; 1.1033x over previous
import functools

import jax
import jax.numpy as jnp
from jax import lax
from jax.experimental import pallas as pl
from jax.experimental.pallas import tpu as pltpu
from jax.experimental.pallas import tpu_sc as plsc

_NC = 2
_NS = 16
_L = 16
_NW = _NC * _NS

_CHUNK = 16384
_UNROLL = 8


def _make_sc_call(total: int, hi: float):
    per_w = total // _NW
    n_chunks = per_w // _CHUNK
    mesh = plsc.VectorSubcoreMesh(
        core_axis_name="c", subcore_axis_name="s",
        num_cores=_NC, num_subcores=_NS)

    @functools.partial(
        pl.kernel,
        out_type=jax.ShapeDtypeStruct((total,), jnp.float32),
        mesh=mesh,
        compiler_params=pltpu.CompilerParams(needs_layout_passes=False),
        scratch_types=[
            pltpu.VMEM((32,), jnp.int32),
            pltpu.VMEM((32,), jnp.int32),
            pltpu.VMEM((_CHUNK,), jnp.float32),
            pltpu.VMEM((_CHUNK,), jnp.float32),
            pltpu.VMEM((_CHUNK,), jnp.float32),
            pltpu.VMEM((_CHUNK,), jnp.float32),
            pltpu.VMEM((_CHUNK,), jnp.float32),
            pltpu.VMEM((_CHUNK,), jnp.float32),
            pltpu.SemaphoreType.DMA,
            pltpu.SemaphoreType.DMA,
            pltpu.SemaphoreType.DMA,
            pltpu.SemaphoreType.DMA,
            pltpu.SemaphoreType.DMA,
            pltpu.SemaphoreType.DMA,
        ],
    )
    def sc_kernel(x_hbm, e_hbm, pm_hbm, ps_hbm, o_hbm,
                  pm_v, ps_v,
                  x0, x1, e0, e1, o0, o1,
                  sx0, sx1, se0, se1, so0, so1):
        wid = lax.axis_index("s") * _NC + lax.axis_index("c")
        base = wid * per_w
        pltpu.sync_copy(pm_hbm, pm_v)
        pltpu.sync_copy(ps_hbm, ps_v)

        xb, eb, ob = (x0, x1), (e0, e1), (o0, o1)
        sx, se, so = (sx0, sx1), (se0, se1), (so0, so1)

        def cin(ci, b):
            off = base + ci * _CHUNK
            return (
                pltpu.make_async_copy(
                    x_hbm.at[pl.ds(off, _CHUNK)], xb[b], sx[b]),
                pltpu.make_async_copy(
                    e_hbm.at[pl.ds(off, _CHUNK)], eb[b], se[b]),
            )

        def cout(ci, b):
            off = base + ci * _CHUNK
            return pltpu.make_async_copy(
                ob[b], o_hbm.at[pl.ds(off, _CHUNK)], so[b])

        def compute(xr, er, outr):
            hi_mask = jnp.int32(-65536)

            @plsc.parallel_loop(0, _CHUNK, step=_L, unroll=_UNROLL)
            def _body(i):
                xv = xr[pl.ds(i, _L)]
                t = xv * hi
                idx = t.astype(jnp.int32)
                gm = plsc.load_gather(pm_v, [idx])
                gs = plsc.load_gather(ps_v, [idx])
                c0m = plsc.bitcast(gm & hi_mask, jnp.float32)
                c1m = plsc.bitcast(gm << 16, jnp.float32)
                c0s = plsc.bitcast(gs & hi_mask, jnp.float32)
                c1s = plsc.bitcast(gs << 16, jnp.float32)
                ev = er[pl.ds(i, _L)]
                outr[pl.ds(i, _L)] = (
                    (c0m + t * c1m) + (c0s + t * c1s) * ev)

        for d in cin(0, 0):
            d.start()

        def outer(g, _):
            for b in range(2):
                ci = 2 * g + b

                @pl.when(ci + 1 < n_chunks)
                def _():
                    for d in cin(ci + 1, 1 - b):
                        d.start()

                for d in cin(ci, b):
                    d.wait()

                @pl.when(ci >= 2)
                def _():
                    cout(ci - 2, b).wait()

                compute(xb[b], eb[b], ob[b])
                cout(ci, b).start()
            return 0

        lax.fori_loop(0, n_chunks // 2, outer, 0)
        cout(n_chunks - 2, 0).wait()
        cout(n_chunks - 1, 1).wait()

    return sc_kernel


def _pack_bf16_pair(c0, c1, pad_to=32):
    hb = lax.bitcast_convert_type(
        c0.astype(jnp.bfloat16), jnp.uint16).astype(jnp.uint32)
    lb = lax.bitcast_convert_type(
        c1.astype(jnp.bfloat16), jnp.uint16).astype(jnp.uint32)
    packed = lax.bitcast_convert_type((hb << 16) | lb, jnp.int32)
    pad = jnp.zeros((pad_to - packed.shape[0],), jnp.int32)
    return jnp.concatenate([packed, pad])


def kernel(x, means, stds, epsilon):
    shape = x.shape
    total = x.size
    num_levels = means.shape[0]
    scale = 1.0 / 4095.0
    m = means.astype(jnp.float32) * scale
    s = stds.astype(jnp.float32) * scale
    zero = jnp.zeros((1,), jnp.float32)
    dm = jnp.concatenate([m[1:] - m[:-1], zero])
    ds = jnp.concatenate([s[1:] - s[:-1], zero])
    f = jnp.arange(num_levels, dtype=jnp.float32)
    pm = _pack_bf16_pair(m - f * dm, dm)
    ps = _pack_bf16_pair(s - f * ds, ds)
    xf = x.reshape(total)
    ef = epsilon.reshape(total)
    out = _make_sc_call(total, float(num_levels - 1))(xf, ef, pm, ps)
    return out.reshape(shape)


# speedup vs baseline: 6.0397x; 2.5155x over previous
import functools

import jax
import jax.numpy as jnp
from jax import lax
from jax.experimental import pallas as pl
from jax.experimental.pallas import tpu as pltpu
from jax.experimental.pallas import tpu_sc as plsc

_NC = 2
_NS = 16
_L = 16
_NW = _NC * _NS

_ROW_CHUNK = 16
_UNROLL = 8


def _make_sc_call(nrows: int, ncols: int, hi: float):
    rows_per_w = nrows // _NW
    n_chunks = rows_per_w // _ROW_CHUNK
    chunk = _ROW_CHUNK * ncols
    mesh = plsc.VectorSubcoreMesh(
        core_axis_name="c", subcore_axis_name="s",
        num_cores=_NC, num_subcores=_NS)

    @functools.partial(
        pl.kernel,
        out_type=jax.ShapeDtypeStruct((nrows, ncols), jnp.float32),
        mesh=mesh,
        compiler_params=pltpu.CompilerParams(
            needs_layout_passes=False, use_tc_tiling_on_sc=True),
        scratch_types=[
            pltpu.VMEM((32,), jnp.int32),
            pltpu.VMEM((32,), jnp.int32),
            pltpu.VMEM((_ROW_CHUNK, ncols), jnp.float32),
            pltpu.VMEM((_ROW_CHUNK, ncols), jnp.float32),
            pltpu.VMEM((_ROW_CHUNK, ncols), jnp.float32),
            pltpu.VMEM((_ROW_CHUNK, ncols), jnp.float32),
            pltpu.VMEM((_ROW_CHUNK, ncols), jnp.float32),
            pltpu.VMEM((_ROW_CHUNK, ncols), jnp.float32),
            pltpu.SemaphoreType.DMA,
            pltpu.SemaphoreType.DMA,
            pltpu.SemaphoreType.DMA,
            pltpu.SemaphoreType.DMA,
            pltpu.SemaphoreType.DMA,
            pltpu.SemaphoreType.DMA,
        ],
    )
    def sc_kernel(x_hbm, e_hbm, pm_hbm, ps_hbm, o_hbm,
                  pm_v, ps_v,
                  x0, x1, e0, e1, o0, o1,
                  sx0, sx1, se0, se1, so0, so1):
        wid = lax.axis_index("s") * _NC + lax.axis_index("c")
        base = wid * rows_per_w
        pltpu.sync_copy(pm_hbm, pm_v)
        pltpu.sync_copy(ps_hbm, ps_v)

        xb, eb, ob = (x0, x1), (e0, e1), (o0, o1)
        sx, se, so = (sx0, sx1), (se0, se1), (so0, so1)

        def cin(ci, b):
            r0 = base + ci * _ROW_CHUNK
            return (
                pltpu.make_async_copy(
                    x_hbm.at[pl.ds(r0, _ROW_CHUNK), :], xb[b], sx[b]),
                pltpu.make_async_copy(
                    e_hbm.at[pl.ds(r0, _ROW_CHUNK), :], eb[b], se[b]),
            )

        def cout(ci, b):
            r0 = base + ci * _ROW_CHUNK
            return pltpu.make_async_copy(
                ob[b], o_hbm.at[pl.ds(r0, _ROW_CHUNK), :], so[b])

        def compute(xr, er, outr):
            hi_mask = jnp.int32(-65536)

            @plsc.parallel_loop(0, chunk, step=_L, unroll=_UNROLL)
            def _body(i):
                r = i // ncols
                c = i - r * ncols
                xv = xr[r, pl.ds(c, _L)]
                t = xv * hi
                idx = t.astype(jnp.int32)
                gm = plsc.load_gather(pm_v, [idx])
                gs = plsc.load_gather(ps_v, [idx])
                c0m = plsc.bitcast(gm & hi_mask, jnp.float32)
                c1m = plsc.bitcast(gm << 16, jnp.float32)
                c0s = plsc.bitcast(gs & hi_mask, jnp.float32)
                c1s = plsc.bitcast(gs << 16, jnp.float32)
                ev = er[r, pl.ds(c, _L)]
                outr[r, pl.ds(c, _L)] = (
                    (c0m + t * c1m) + (c0s + t * c1s) * ev)

        for d in cin(0, 0):
            d.start()

        def outer(g, _):
            for b in range(2):
                ci = 2 * g + b

                @pl.when(ci + 1 < n_chunks)
                def _():
                    for d in cin(ci + 1, 1 - b):
                        d.start()

                for d in cin(ci, b):
                    d.wait()

                @pl.when(ci >= 2)
                def _():
                    cout(ci - 2, b).wait()

                compute(xb[b], eb[b], ob[b])
                cout(ci, b).start()
            return 0

        lax.fori_loop(0, n_chunks // 2, outer, 0)
        cout(n_chunks - 2, 0).wait()
        cout(n_chunks - 1, 1).wait()

    return sc_kernel


def _pack_bf16_pair(c0, c1, pad_to=32):
    hb = lax.bitcast_convert_type(
        c0.astype(jnp.bfloat16), jnp.uint16).astype(jnp.uint32)
    lb = lax.bitcast_convert_type(
        c1.astype(jnp.bfloat16), jnp.uint16).astype(jnp.uint32)
    packed = lax.bitcast_convert_type((hb << 16) | lb, jnp.int32)
    pad = jnp.zeros((pad_to - packed.shape[0],), jnp.int32)
    return jnp.concatenate([packed, pad])


def kernel(x, means, stds, epsilon):
    nrows, ncols = x.shape
    num_levels = means.shape[0]
    scale = 1.0 / 4095.0
    m = means.astype(jnp.float32) * scale
    s = stds.astype(jnp.float32) * scale
    zero = jnp.zeros((1,), jnp.float32)
    dm = jnp.concatenate([m[1:] - m[:-1], zero])
    ds = jnp.concatenate([s[1:] - s[:-1], zero])
    f = jnp.arange(num_levels, dtype=jnp.float32)
    pm = _pack_bf16_pair(m - f * dm, dm)
    ps = _pack_bf16_pair(s - f * ds, ds)
    return _make_sc_call(nrows, ncols, float(num_levels - 1))(
        x, epsilon, pm, ps)


# speedup vs baseline: 6.3197x; 1.0464x over previous
import functools

import jax
import jax.numpy as jnp
from jax import lax
from jax.experimental import pallas as pl
from jax.experimental.pallas import tpu as pltpu
from jax.experimental.pallas import tpu_sc as plsc

_NC = 2
_NS = 16
_L = 16
_NW = _NC * _NS

_ROW_CHUNK = 16
_UNROLL = 8


def _make_sc_call(nrows: int, ncols: int, hi: float):
    rows_per_w = nrows // _NW
    n_chunks = rows_per_w // _ROW_CHUNK
    chunk = _ROW_CHUNK * ncols
    mesh = plsc.VectorSubcoreMesh(
        core_axis_name="c", subcore_axis_name="s",
        num_cores=_NC, num_subcores=_NS)

    @functools.partial(
        pl.kernel,
        out_type=jax.ShapeDtypeStruct((nrows, ncols), jnp.float32),
        mesh=mesh,
        compiler_params=pltpu.CompilerParams(
            needs_layout_passes=False, use_tc_tiling_on_sc=True),
        scratch_types=[
            pltpu.VMEM((32,), jnp.int32),
            pltpu.VMEM((32,), jnp.int32),
            pltpu.VMEM((_ROW_CHUNK, ncols), jnp.float32),
            pltpu.VMEM((_ROW_CHUNK, ncols), jnp.float32),
            pltpu.VMEM((_ROW_CHUNK, ncols), jnp.float32),
            pltpu.VMEM((_ROW_CHUNK, ncols), jnp.float32),
            pltpu.VMEM((_ROW_CHUNK, ncols), jnp.float32),
            pltpu.VMEM((_ROW_CHUNK, ncols), jnp.float32),
            pltpu.SemaphoreType.DMA,
            pltpu.SemaphoreType.DMA,
            pltpu.SemaphoreType.DMA,
            pltpu.SemaphoreType.DMA,
            pltpu.SemaphoreType.DMA,
            pltpu.SemaphoreType.DMA,
        ],
    )
    def sc_kernel(x_hbm, e_hbm, pm_hbm, ps_hbm, o_hbm,
                  pm_v, ps_v,
                  x0, x1, e0, e1, o0, o1,
                  sx0, sx1, se0, se1, so0, so1):
        wid = lax.axis_index("s") * _NC + lax.axis_index("c")
        base = wid * rows_per_w
        pltpu.sync_copy(pm_hbm, pm_v)
        pltpu.sync_copy(ps_hbm, ps_v)

        xb, eb, ob = (x0, x1), (e0, e1), (o0, o1)
        sx, se, so = (sx0, sx1), (se0, se1), (so0, so1)

        def cin(ci, b):
            r0 = base + ci * _ROW_CHUNK
            return (
                pltpu.make_async_copy(
                    x_hbm.at[pl.ds(r0, _ROW_CHUNK), :], xb[b], sx[b]),
                pltpu.make_async_copy(
                    e_hbm.at[pl.ds(r0, _ROW_CHUNK), :], eb[b], se[b]),
            )

        def cout(ci, b):
            r0 = base + ci * _ROW_CHUNK
            return pltpu.make_async_copy(
                ob[b], o_hbm.at[pl.ds(r0, _ROW_CHUNK), :], so[b])

        def compute(xr, er, outr):
            @plsc.parallel_loop(0, chunk, step=_L, unroll=_UNROLL)
            def _body(i):
                r = i // ncols
                c = i - r * ncols
                xv = xr[r, pl.ds(c, _L)]
                t = xv * hi
                idx = t.astype(jnp.int32)
                gm = plsc.load_gather(pm_v, [idx])
                gs = plsc.load_gather(ps_v, [idx])
                c0m = plsc.bitcast(gm, jnp.float32)
                c1m = plsc.bitcast(gm << 16, jnp.float32)
                c0s = plsc.bitcast(gs, jnp.float32)
                c1s = plsc.bitcast(gs << 16, jnp.float32)
                ev = er[r, pl.ds(c, _L)]
                outr[r, pl.ds(c, _L)] = (
                    (c0m + t * c1m) + (c0s + t * c1s) * ev)

        for d in cin(0, 0):
            d.start()

        def outer(g, _):
            for b in range(2):
                ci = 2 * g + b

                @pl.when(ci + 1 < n_chunks)
                def _():
                    for d in cin(ci + 1, 1 - b):
                        d.start()

                for d in cin(ci, b):
                    d.wait()

                @pl.when(ci >= 2)
                def _():
                    cout(ci - 2, b).wait()

                compute(xb[b], eb[b], ob[b])
                cout(ci, b).start()
            return 0

        lax.fori_loop(0, n_chunks // 2, outer, 0)
        cout(n_chunks - 2, 0).wait()
        cout(n_chunks - 1, 1).wait()

    return sc_kernel


def _pack_bf16_pair(c0, c1, pad_to=32):
    lb = lax.bitcast_convert_type(
        c1.astype(jnp.bfloat16), jnp.uint16).astype(jnp.uint32)
    b = lax.bitcast_convert_type(c0.astype(jnp.float32), jnp.uint32)
    cand0 = (b & jnp.uint32(0xFFFF0000)) | lb
    cands = jnp.stack([cand0 - jnp.uint32(0x10000), cand0,
                       cand0 + jnp.uint32(0x10000)])
    vals = lax.bitcast_convert_type(cands, jnp.float32)
    best = jnp.argmin(jnp.abs(vals - c0[None, :]), axis=0)
    packed = lax.bitcast_convert_type(
        jnp.take_along_axis(cands, best[None, :], axis=0)[0], jnp.int32)
    pad = jnp.zeros((pad_to - packed.shape[0],), jnp.int32)
    return jnp.concatenate([packed, pad])


def kernel(x, means, stds, epsilon):
    nrows, ncols = x.shape
    num_levels = means.shape[0]
    scale = 1.0 / 4095.0
    m = means.astype(jnp.float32) * scale
    s = stds.astype(jnp.float32) * scale
    zero = jnp.zeros((1,), jnp.float32)
    dm = jnp.concatenate([m[1:] - m[:-1], zero])
    ds = jnp.concatenate([s[1:] - s[:-1], zero])
    f = jnp.arange(num_levels, dtype=jnp.float32)
    pm = _pack_bf16_pair(m - f * dm, dm)
    ps = _pack_bf16_pair(s - f * ds, ds)
    return _make_sc_call(nrows, ncols, float(num_levels - 1))(
        x, epsilon, pm, ps)


# speedup vs baseline: 6.4093x; 1.0142x over previous
import functools

import jax
import jax.numpy as jnp
from jax import lax
from jax.experimental import pallas as pl
from jax.experimental.pallas import tpu as pltpu
from jax.experimental.pallas import tpu_sc as plsc

_NC = 2
_NS = 16
_L = 16
_NW = _NC * _NS

_ROW_CHUNK = 16
_UNROLL = 16


def _make_sc_call(nrows: int, ncols: int, hi: float):
    rows_per_w = nrows // _NW
    n_chunks = rows_per_w // _ROW_CHUNK
    chunk = _ROW_CHUNK * ncols
    mesh = plsc.VectorSubcoreMesh(
        core_axis_name="c", subcore_axis_name="s",
        num_cores=_NC, num_subcores=_NS)

    @functools.partial(
        pl.kernel,
        out_type=jax.ShapeDtypeStruct((nrows, ncols), jnp.float32),
        mesh=mesh,
        compiler_params=pltpu.CompilerParams(
            needs_layout_passes=False, use_tc_tiling_on_sc=True),
        scratch_types=[
            pltpu.VMEM((32,), jnp.int32),
            pltpu.VMEM((32,), jnp.int32),
            pltpu.VMEM((_ROW_CHUNK, ncols), jnp.float32),
            pltpu.VMEM((_ROW_CHUNK, ncols), jnp.float32),
            pltpu.VMEM((_ROW_CHUNK, ncols), jnp.float32),
            pltpu.VMEM((_ROW_CHUNK, ncols), jnp.float32),
            pltpu.VMEM((_ROW_CHUNK, ncols), jnp.float32),
            pltpu.VMEM((_ROW_CHUNK, ncols), jnp.float32),
            pltpu.SemaphoreType.DMA,
            pltpu.SemaphoreType.DMA,
            pltpu.SemaphoreType.DMA,
            pltpu.SemaphoreType.DMA,
            pltpu.SemaphoreType.DMA,
            pltpu.SemaphoreType.DMA,
        ],
    )
    def sc_kernel(x_hbm, e_hbm, pm_hbm, ps_hbm, o_hbm,
                  pm_v, ps_v,
                  x0, x1, e0, e1, o0, o1,
                  sx0, sx1, se0, se1, so0, so1):
        wid = lax.axis_index("s") * _NC + lax.axis_index("c")
        base = wid * rows_per_w
        pltpu.sync_copy(pm_hbm, pm_v)
        pltpu.sync_copy(ps_hbm, ps_v)

        xb, eb, ob = (x0, x1), (e0, e1), (o0, o1)
        sx, se, so = (sx0, sx1), (se0, se1), (so0, so1)

        def cin(ci, b):
            r0 = base + ci * _ROW_CHUNK
            return (
                pltpu.make_async_copy(
                    x_hbm.at[pl.ds(r0, _ROW_CHUNK), :], xb[b], sx[b]),
                pltpu.make_async_copy(
                    e_hbm.at[pl.ds(r0, _ROW_CHUNK), :], eb[b], se[b]),
            )

        def cout(ci, b):
            r0 = base + ci * _ROW_CHUNK
            return pltpu.make_async_copy(
                ob[b], o_hbm.at[pl.ds(r0, _ROW_CHUNK), :], so[b])

        def compute(xr, er, outr):
            @plsc.parallel_loop(0, chunk, step=_L, unroll=_UNROLL)
            def _body(i):
                r = i // ncols
                c = i - r * ncols
                xv = xr[r, pl.ds(c, _L)]
                t = xv * hi
                idx = t.astype(jnp.int32)
                gm = plsc.load_gather(pm_v, [idx])
                gs = plsc.load_gather(ps_v, [idx])
                c0m = plsc.bitcast(gm, jnp.float32)
                c1m = plsc.bitcast(gm << 16, jnp.float32)
                c0s = plsc.bitcast(gs, jnp.float32)
                c1s = plsc.bitcast(gs << 16, jnp.float32)
                ev = er[r, pl.ds(c, _L)]
                outr[r, pl.ds(c, _L)] = (
                    (c0m + t * c1m) + (c0s + t * c1s) * ev)

        for d in cin(0, 0):
            d.start()

        def outer(g, _):
            for b in range(2):
                ci = 2 * g + b

                @pl.when(ci + 1 < n_chunks)
                def _():
                    for d in cin(ci + 1, 1 - b):
                        d.start()

                for d in cin(ci, b):
                    d.wait()

                @pl.when(ci >= 2)
                def _():
                    cout(ci - 2, b).wait()

                compute(xb[b], eb[b], ob[b])
                cout(ci, b).start()
            return 0

        lax.fori_loop(0, n_chunks // 2, outer, 0)
        cout(n_chunks - 2, 0).wait()
        cout(n_chunks - 1, 1).wait()

    return sc_kernel


def _pack_bf16_pair(c0, c1, pad_to=32):
    lb = lax.bitcast_convert_type(
        c1.astype(jnp.bfloat16), jnp.uint16).astype(jnp.uint32)
    b = lax.bitcast_convert_type(c0.astype(jnp.float32), jnp.uint32)
    cand0 = (b & jnp.uint32(0xFFFF0000)) | lb
    cands = jnp.stack([cand0 - jnp.uint32(0x10000), cand0,
                       cand0 + jnp.uint32(0x10000)])
    vals = lax.bitcast_convert_type(cands, jnp.float32)
    best = jnp.argmin(jnp.abs(vals - c0[None, :]), axis=0)
    packed = lax.bitcast_convert_type(
        jnp.take_along_axis(cands, best[None, :], axis=0)[0], jnp.int32)
    pad = jnp.zeros((pad_to - packed.shape[0],), jnp.int32)
    return jnp.concatenate([packed, pad])


def kernel(x, means, stds, epsilon):
    nrows, ncols = x.shape
    num_levels = means.shape[0]
    scale = 1.0 / 4095.0
    m = means.astype(jnp.float32) * scale
    s = stds.astype(jnp.float32) * scale
    zero = jnp.zeros((1,), jnp.float32)
    dm = jnp.concatenate([m[1:] - m[:-1], zero])
    ds = jnp.concatenate([s[1:] - s[:-1], zero])
    f = jnp.arange(num_levels, dtype=jnp.float32)
    pm = _pack_bf16_pair(m - f * dm, dm)
    ps = _pack_bf16_pair(s - f * ds, ds)
    return _make_sc_call(nrows, ncols, float(num_levels - 1))(
        x, epsilon, pm, ps)
